# baseline (speedup 1.0000x reference)
import functools

import jax
import jax.numpy as jnp
from jax import lax
from jax.experimental import pallas as pl
from jax.experimental.pallas import tpu as pltpu
from jax.experimental.pallas import tpu_sc as plsc

N = 100000
X = 128
H = 128

_NC = 2
_NS = 16
_NW = _NC * _NS
_CHUNK = 128
_CPW = 50
_PAD2N = _NW * _CPW * _CHUNK


def _gather_body(h_hbm, c_hbm, idx_hbm, hout_hbm, cout_hbm,
                 idx_all, h_v0, h_v1, c_v0, c_v1, gsem0, gsem1, ssem0, ssem1):
    wid = lax.axis_index("s") * _NC + lax.axis_index("c")
    wbase = wid * _CPW * _CHUNK
    pltpu.sync_copy(idx_hbm.at[pl.ds(wbase, _CPW * _CHUNK)], idx_all)

    h_v = (h_v0, h_v1)
    c_v = (c_v0, c_v1)
    gsem = (gsem0, gsem1)
    ssem = (ssem0, ssem1)

    def fire_gather(j, s):
        idx_v = idx_all.at[pl.ds(j * _CHUNK, _CHUNK)]
        pltpu.async_copy(h_hbm.at[idx_v], h_v[s], gsem[s])
        pltpu.async_copy(c_hbm.at[idx_v], c_v[s], gsem[s])

    def fire_scatter(j, s):
        base = wbase + j * _CHUNK
        pltpu.async_copy(h_v[s], hout_hbm.at[pl.ds(base, _CHUNK)], ssem[s])
        pltpu.async_copy(c_v[s], cout_hbm.at[pl.ds(base, _CHUNK)], ssem[s])

    def drain_gather(s):
        pltpu.make_async_copy(hout_hbm.at[pl.ds(0, _CHUNK)], h_v[s], gsem[s]).wait()
        pltpu.make_async_copy(cout_hbm.at[pl.ds(0, _CHUNK)], c_v[s], gsem[s]).wait()

    def drain_scatter(s):
        pltpu.make_async_copy(h_v[s], hout_hbm.at[pl.ds(0, _CHUNK)], ssem[s]).wait()
        pltpu.make_async_copy(c_v[s], cout_hbm.at[pl.ds(0, _CHUNK)], ssem[s]).wait()

    fire_gather(0, 0)

    def body(i, carry):
        for b in (0, 1):
            j = 2 * i + b
            @pl.when(j >= 1)
            def _():
                drain_scatter(1 - b)

            @pl.when(j + 1 < _CPW)
            def _():
                fire_gather(j + 1, 1 - b)

            drain_gather(b)
            fire_scatter(j, b)
        return carry

    lax.fori_loop(0, _CPW // 2, body, 0)
    drain_scatter((_CPW - 1) % 2)


@functools.cache
def _make_sc_gather():
    return functools.partial(
        pl.kernel,
        mesh=plsc.VectorSubcoreMesh(core_axis_name="c", subcore_axis_name="s"),
        out_type=(
            jax.ShapeDtypeStruct((_PAD2N, H), jnp.float32),
            jax.ShapeDtypeStruct((_PAD2N, H), jnp.float32),
        ),
        scratch_types=[
            pltpu.VMEM((_CPW * _CHUNK,), jnp.int32),
            pltpu.VMEM((_CHUNK, H), jnp.float32),
            pltpu.VMEM((_CHUNK, H), jnp.float32),
            pltpu.VMEM((_CHUNK, H), jnp.float32),
            pltpu.VMEM((_CHUNK, H), jnp.float32),
            pltpu.SemaphoreType.DMA,
            pltpu.SemaphoreType.DMA,
            pltpu.SemaphoreType.DMA,
            pltpu.SemaphoreType.DMA,
        ],
    )(_gather_body)


_BN = 2000


def _dense_body(x_ref, hcat_ref, ccat_ref, tf_ref,
                wx_ref, wh_ref, wf_ref, biou_ref, bious_ref, bf_ref,
                h_out_ref, c_out_ref):
    x = x_ref[...]
    hcat = hcat_ref[...]
    iou_both = (
        jnp.dot(x, wx_ref[...], preferred_element_type=jnp.float32)
        + jnp.dot(hcat, wh_ref[...], preferred_element_type=jnp.float32)
    )
    f = jax.nn.sigmoid(
        jnp.dot(hcat, wf_ref[...], preferred_element_type=jnp.float32)
        + bf_ref[...]
    )
    ccat = ccat_ref[...]
    c1 = ccat[:, :H]
    c2 = ccat[:, H:]
    c_n = f[:, :H] * c1 + f[:, H:2 * H] * c2
    c_sm = f[:, 2 * H:3 * H] * c1 + f[:, 3 * H:] * c2
    iou_n = iou_both[:, :3 * H] + biou_ref[...]
    iou_sm = iou_both[:, 3 * H:] + bious_ref[...]
    tm = tf_ref[...]
    iou = iou_n + tm * (iou_sm - iou_n)
    c_r = c_n + tm * (c_sm - c_n)
    c_out = jax.nn.sigmoid(iou[:, :H]) * jnp.tanh(iou[:, 2 * H:]) + c_r
    c_out_ref[...] = c_out
    h_out_ref[...] = jax.nn.sigmoid(iou[:, H:2 * H]) * jnp.tanh(c_out)


def _dense_call(x, hcat, ccat, tf, wx, wh, wf, biou, bious, bf):
    grid = (N // _BN,)
    row_spec = lambda w: pl.BlockSpec((_BN, w), lambda i: (i, 0))
    full_spec = lambda a, b: pl.BlockSpec((a, b), lambda i: (0, 0))
    return pl.pallas_call(
        _dense_body,
        grid=grid,
        in_specs=[
            row_spec(X),
            row_spec(2 * H),
            row_spec(2 * H),
            row_spec(1),
            full_spec(X, 6 * H),
            full_spec(2 * H, 6 * H),
            full_spec(2 * H, 4 * H),
            full_spec(1, 3 * H),
            full_spec(1, 3 * H),
            full_spec(1, 4 * H),
        ],
        out_specs=[row_spec(H), row_spec(H)],
        out_shape=[
            jax.ShapeDtypeStruct((N, H), jnp.float32),
            jax.ShapeDtypeStruct((N, H), jnp.float32),
        ],
    )(x, hcat, ccat, tf, wx, wh, wf, biou, bious, bf)


def kernel(x, h_src, c_src, child_idx, t, W_iou, U_iou, b_iou, U_f_w, U_f_b,
           W_iou_s, U_iou_s, b_iou_s, U_f_s_w, U_f_s_b):
    idx = jnp.concatenate(
        [child_idx.reshape(-1),
         jnp.zeros((_PAD2N - 2 * N,), dtype=jnp.int32)])

    h_mail, c_mail = _make_sc_gather()(h_src, c_src, idx)
    hcat = h_mail.reshape(_PAD2N // 2, 2 * H)
    ccat = c_mail.reshape(_PAD2N // 2, 2 * H)

    wx = jnp.concatenate([W_iou, W_iou_s], axis=1)
    wh = jnp.concatenate(
        [U_iou, jnp.concatenate([U_iou_s, U_iou_s], axis=0)], axis=1)
    z = jnp.zeros((H, H), dtype=jnp.float32)
    ufs_bd = jnp.block([[U_f_s_w, z], [z, U_f_s_w]])
    wf = jnp.concatenate([U_f_w, ufs_bd], axis=1)
    bf = jnp.concatenate([U_f_b, U_f_s_b, U_f_s_b]).reshape(1, 4 * H)
    tf = (t == 1).astype(jnp.float32).reshape(N, 1)

    h_out, c_out = _dense_call(x, hcat, ccat, tf, wx, wh, wf,
                               b_iou, b_iou_s, bf)
    return (h_out, c_out)


# speedup vs baseline: 1.2755x; 1.2755x over previous
import functools

import jax
import jax.numpy as jnp
from jax import lax
from jax.experimental import pallas as pl
from jax.experimental.pallas import tpu as pltpu
from jax.experimental.pallas import tpu_sc as plsc

N = 100000
X = 128
H = 128

_NC = 2
_NS = 16
_NW = _NC * _NS
_CHUNK = 128
_CPW = 50
_PAD2N = _NW * _CPW * _CHUNK
_PADN2 = _PAD2N // 2


def _gather_body(h_hbm, c_hbm, idx_hbm, hout_hbm, cout_hbm,
                 idx_all, h_v, c_v, gsem):
    wid = lax.axis_index("s") * _NC + lax.axis_index("c")
    wbase = wid * _CPW * _CHUNK
    pltpu.sync_copy(idx_hbm.at[pl.ds(wbase, _CPW * _CHUNK)], idx_all)

    def chunk(j, carry):
        base = wbase + j * _CHUNK
        idx_v = idx_all.at[pl.ds(j * _CHUNK, _CHUNK)]
        cph = pltpu.async_copy(h_hbm.at[idx_v], h_v, gsem)
        cpc = pltpu.async_copy(c_hbm.at[idx_v], c_v, gsem)
        cph.wait()
        cpc.wait()
        pltpu.sync_copy(h_v, hout_hbm.at[pl.ds(base, _CHUNK)])
        pltpu.sync_copy(c_v, cout_hbm.at[pl.ds(base, _CHUNK)])
        return carry

    lax.fori_loop(0, _CPW, chunk, 0)


@functools.cache
def _make_sc_gather():
    return functools.partial(
        pl.kernel,
        mesh=plsc.VectorSubcoreMesh(core_axis_name="c", subcore_axis_name="s"),
        out_type=(
            jax.ShapeDtypeStruct((_PAD2N, H), jnp.float32),
            jax.ShapeDtypeStruct((_PAD2N, H), jnp.float32),
        ),
        scratch_types=[
            pltpu.VMEM((_CPW * _CHUNK,), jnp.int32),
            pltpu.VMEM((_CHUNK, H), jnp.float32),
            pltpu.VMEM((_CHUNK, H), jnp.float32),
            pltpu.SemaphoreType.DMA,
        ],
    )(_gather_body)


_BN = 800


def _dense_body(x_ref, h1_ref, h2_ref, c1_ref, c2_ref, tf_ref,
                wx_ref, wh_ref, wf_ref, biou_ref, bious_ref, bf_ref,
                h_out_ref, c_out_ref):
    x = x_ref[...]
    h1 = h1_ref[...]
    h2 = h2_ref[...]
    wh = wh_ref[...]
    wf = wf_ref[...]
    iou_both = (
        jnp.dot(x, wx_ref[...], preferred_element_type=jnp.float32)
        + jnp.dot(h1, wh[:H], preferred_element_type=jnp.float32)
        + jnp.dot(h2, wh[H:], preferred_element_type=jnp.float32)
    )
    f = jax.nn.sigmoid(
        jnp.dot(h1, wf[:H], preferred_element_type=jnp.float32)
        + jnp.dot(h2, wf[H:], preferred_element_type=jnp.float32)
        + bf_ref[...]
    )
    c1 = c1_ref[...]
    c2 = c2_ref[...]
    c_n = f[:, :H] * c1 + f[:, H:2 * H] * c2
    c_sm = f[:, 2 * H:3 * H] * c1 + f[:, 3 * H:] * c2
    iou_n = iou_both[:, :3 * H] + biou_ref[...]
    iou_sm = iou_both[:, 3 * H:] + bious_ref[...]
    tm = tf_ref[...]
    iou = iou_n + tm * (iou_sm - iou_n)
    c_r = c_n + tm * (c_sm - c_n)
    c_out = jax.nn.sigmoid(iou[:, :H]) * jnp.tanh(iou[:, 2 * H:]) + c_r
    c_out_ref[...] = c_out
    h_out_ref[...] = jax.nn.sigmoid(iou[:, H:2 * H]) * jnp.tanh(c_out)


def _dense_call(x, h_mail, c_mail, tf, wx, wh, wf, biou, bious, bf):
    grid = (N // _BN,)
    off = _PADN2 // _BN
    row_spec = lambda w: pl.BlockSpec((_BN, w), lambda i: (i, 0))
    off_spec = pl.BlockSpec((_BN, H), lambda i: (i + off, 0))
    full_spec = lambda a, b: pl.BlockSpec((a, b), lambda i: (0, 0))
    return pl.pallas_call(
        _dense_body,
        grid=grid,
        in_specs=[
            row_spec(X),
            row_spec(H),
            off_spec,
            row_spec(H),
            off_spec,
            row_spec(1),
            full_spec(X, 6 * H),
            full_spec(2 * H, 6 * H),
            full_spec(2 * H, 4 * H),
            full_spec(1, 3 * H),
            full_spec(1, 3 * H),
            full_spec(1, 4 * H),
        ],
        out_specs=[row_spec(H), row_spec(H)],
        out_shape=[
            jax.ShapeDtypeStruct((N, H), jnp.float32),
            jax.ShapeDtypeStruct((N, H), jnp.float32),
        ],
    )(x, h_mail, h_mail, c_mail, c_mail, tf, wx, wh, wf, biou, bious, bf)


def kernel(x, h_src, c_src, child_idx, t, W_iou, U_iou, b_iou, U_f_w, U_f_b,
           W_iou_s, U_iou_s, b_iou_s, U_f_s_w, U_f_s_b):
    zpad = jnp.zeros((_PADN2 - N,), dtype=jnp.int32)
    idx = jnp.concatenate([child_idx[:, 0], zpad, child_idx[:, 1], zpad])

    h_mail, c_mail = _make_sc_gather()(h_src, c_src, idx)

    wx = jnp.concatenate([W_iou, W_iou_s], axis=1)
    wh = jnp.concatenate(
        [U_iou, jnp.concatenate([U_iou_s, U_iou_s], axis=0)], axis=1)
    z = jnp.zeros((H, H), dtype=jnp.float32)
    ufs_bd = jnp.block([[U_f_s_w, z], [z, U_f_s_w]])
    wf = jnp.concatenate([U_f_w, ufs_bd], axis=1)
    bf = jnp.concatenate([U_f_b, U_f_s_b, U_f_s_b]).reshape(1, 4 * H)
    tf = (t == 1).astype(jnp.float32).reshape(N, 1)

    h_out, c_out = _dense_call(x, h_mail, c_mail, tf, wx, wh, wf,
                               b_iou, b_iou_s, bf)
    return (h_out, c_out)


# speedup vs baseline: 1.3271x; 1.0404x over previous
import functools

import jax
import jax.numpy as jnp
from jax import lax
from jax.experimental import pallas as pl
from jax.experimental.pallas import tpu as pltpu
from jax.experimental.pallas import tpu_sc as plsc

N = 100000
X = 128
H = 128

_NC = 2
_NS = 16
_NW = _NC * _NS
_CHUNK = 128
_CPW = 50
_PAD2N = _NW * _CPW * _CHUNK
_PADN2 = _PAD2N // 2


_NBUF = 2


def _gather_body(h_hbm, c_hbm, idx_hbm, hout_hbm, cout_hbm,
                 idx_all, h_v, c_v, gs0, gs1, ws0, ws1):
    wid = lax.axis_index("s") * _NC + lax.axis_index("c")
    wbase = wid * _CPW * _CHUNK
    gsems = (gs0, gs1)
    wsems = (ws0, ws1)
    pltpu.sync_copy(idx_hbm.at[pl.ds(wbase, _CPW * _CHUNK)], idx_all)

    def bufs(b):
        return (h_v.at[pl.ds(b * _CHUNK, _CHUNK)],
                c_v.at[pl.ds(b * _CHUNK, _CHUNK)])

    def issue_gather(j, b):
        idx_v = idx_all.at[pl.ds(j * _CHUNK, _CHUNK)]
        hbuf, cbuf = bufs(b)
        pltpu.async_copy(h_hbm.at[idx_v], hbuf, gsems[b])
        pltpu.async_copy(c_hbm.at[idx_v], cbuf, gsems[b])

    for b in range(_NBUF):
        issue_gather(b, b)

    def outer(g, carry):
        for b in range(_NBUF):
            j = g * _NBUF + b
            hbuf, cbuf = bufs(b)
            idx_v = idx_all.at[pl.ds(j * _CHUNK, _CHUNK)]
            pltpu.make_async_copy(h_hbm.at[idx_v], hbuf, gsems[b]).wait()
            pltpu.make_async_copy(c_hbm.at[idx_v], cbuf, gsems[b]).wait()
            base = wbase + j * _CHUNK
            cph = pltpu.async_copy(hbuf, hout_hbm.at[pl.ds(base, _CHUNK)],
                                   wsems[b])
            cpc = pltpu.async_copy(cbuf, cout_hbm.at[pl.ds(base, _CHUNK)],
                                   wsems[b])
            cph.wait()
            cpc.wait()

            @pl.when(j + _NBUF < _CPW)
            def _():
                issue_gather(j + _NBUF, b)
        return carry

    lax.fori_loop(0, _CPW // _NBUF, outer, 0)


@functools.cache
def _make_sc_gather():
    return functools.partial(
        pl.kernel,
        mesh=plsc.VectorSubcoreMesh(core_axis_name="c", subcore_axis_name="s"),
        out_type=(
            jax.ShapeDtypeStruct((_PAD2N, H), jnp.float32),
            jax.ShapeDtypeStruct((_PAD2N, H), jnp.float32),
        ),
        scratch_types=[
            pltpu.VMEM((_CPW * _CHUNK,), jnp.int32),
            pltpu.VMEM((_NBUF * _CHUNK, H), jnp.float32),
            pltpu.VMEM((_NBUF * _CHUNK, H), jnp.float32),
            pltpu.SemaphoreType.DMA,
            pltpu.SemaphoreType.DMA,
            pltpu.SemaphoreType.DMA,
            pltpu.SemaphoreType.DMA,
        ],
    )(_gather_body)


_BN = 800


def _dense_body(x_ref, h1_ref, h2_ref, c1_ref, c2_ref, tf_ref,
                wx_ref, wh_ref, wf_ref, biou_ref, bious_ref, bf_ref,
                h_out_ref, c_out_ref):
    x = x_ref[...]
    h1 = h1_ref[...]
    h2 = h2_ref[...]
    wh = wh_ref[...]
    wf = wf_ref[...]
    iou_both = (
        jnp.dot(x, wx_ref[...], preferred_element_type=jnp.float32)
        + jnp.dot(h1, wh[:H], preferred_element_type=jnp.float32)
        + jnp.dot(h2, wh[H:], preferred_element_type=jnp.float32)
    )
    f = jax.nn.sigmoid(
        jnp.dot(h1, wf[:H], preferred_element_type=jnp.float32)
        + jnp.dot(h2, wf[H:], preferred_element_type=jnp.float32)
        + bf_ref[...]
    )
    c1 = c1_ref[...]
    c2 = c2_ref[...]
    c_n = f[:, :H] * c1 + f[:, H:2 * H] * c2
    c_sm = f[:, 2 * H:3 * H] * c1 + f[:, 3 * H:] * c2
    iou_n = iou_both[:, :3 * H] + biou_ref[...]
    iou_sm = iou_both[:, 3 * H:] + bious_ref[...]
    tm = tf_ref[...]
    iou = iou_n + tm * (iou_sm - iou_n)
    c_r = c_n + tm * (c_sm - c_n)
    c_out = jax.nn.sigmoid(iou[:, :H]) * jnp.tanh(iou[:, 2 * H:]) + c_r
    c_out_ref[...] = c_out
    h_out_ref[...] = jax.nn.sigmoid(iou[:, H:2 * H]) * jnp.tanh(c_out)


def _dense_call(x, h_mail, c_mail, tf, wx, wh, wf, biou, bious, bf):
    grid = (N // _BN,)
    off = _PADN2 // _BN
    row_spec = lambda w: pl.BlockSpec((_BN, w), lambda i: (i, 0))
    off_spec = pl.BlockSpec((_BN, H), lambda i: (i + off, 0))
    full_spec = lambda a, b: pl.BlockSpec((a, b), lambda i: (0, 0))
    return pl.pallas_call(
        _dense_body,
        grid=grid,
        in_specs=[
            row_spec(X),
            row_spec(H),
            off_spec,
            row_spec(H),
            off_spec,
            row_spec(1),
            full_spec(X, 6 * H),
            full_spec(2 * H, 6 * H),
            full_spec(2 * H, 4 * H),
            full_spec(1, 3 * H),
            full_spec(1, 3 * H),
            full_spec(1, 4 * H),
        ],
        out_specs=[row_spec(H), row_spec(H)],
        out_shape=[
            jax.ShapeDtypeStruct((N, H), jnp.float32),
            jax.ShapeDtypeStruct((N, H), jnp.float32),
        ],
    )(x, h_mail, h_mail, c_mail, c_mail, tf, wx, wh, wf, biou, bious, bf)


def kernel(x, h_src, c_src, child_idx, t, W_iou, U_iou, b_iou, U_f_w, U_f_b,
           W_iou_s, U_iou_s, b_iou_s, U_f_s_w, U_f_s_b):
    zpad = jnp.zeros((_PADN2 - N,), dtype=jnp.int32)
    idx = jnp.concatenate([child_idx[:, 0], zpad, child_idx[:, 1], zpad])

    h_mail, c_mail = _make_sc_gather()(h_src, c_src, idx)

    wx = jnp.concatenate([W_iou, W_iou_s], axis=1)
    wh = jnp.concatenate(
        [U_iou, jnp.concatenate([U_iou_s, U_iou_s], axis=0)], axis=1)
    z = jnp.zeros((H, H), dtype=jnp.float32)
    ufs_bd = jnp.block([[U_f_s_w, z], [z, U_f_s_w]])
    wf = jnp.concatenate([U_f_w, ufs_bd], axis=1)
    bf = jnp.concatenate([U_f_b, U_f_s_b, U_f_s_b]).reshape(1, 4 * H)
    tf = (t == 1).astype(jnp.float32).reshape(N, 1)

    h_out, c_out = _dense_call(x, h_mail, c_mail, tf, wx, wh, wf,
                               b_iou, b_iou_s, bf)
    return (h_out, c_out)


# speedup vs baseline: 1.4044x; 1.0582x over previous
import functools

import jax
import jax.numpy as jnp
from jax import lax
from jax.experimental import pallas as pl
from jax.experimental.pallas import tpu as pltpu
from jax.experimental.pallas import tpu_sc as plsc

N = 100000
X = 128
H = 128

_K = 5
_NSLAB = N // _K
_CHUNK = 128
_NC = 2
_NSUB = 16
_GPW = 16
_CPW = 10
_NSP = _GPW * _CPW * _CHUNK
_NBUF = 2


def _gather_body(h_hbm, c_hbm, idx0_hbm, idx1_hbm,
                 h0_out, h1_out, c0_out, c1_out,
                 idx_v, h_v, c_v, gs0, gs1, ws0, ws1):
    wid = lax.axis_index("s") * _NC + lax.axis_index("c")
    gsems = (gs0, gs1)
    wsems = (ws0, ws1)

    def run(idx_hbm, hout_hbm, cout_hbm, lw):
        wbase = lw * _CPW * _CHUNK
        pltpu.sync_copy(idx_hbm.at[pl.ds(wbase, _CPW * _CHUNK)], idx_v)

        def bufs(b):
            return (h_v.at[pl.ds(b * _CHUNK, _CHUNK)],
                    c_v.at[pl.ds(b * _CHUNK, _CHUNK)])

        def issue_gather(j, b):
            iv = idx_v.at[pl.ds(j * _CHUNK, _CHUNK)]
            hbuf, cbuf = bufs(b)
            pltpu.async_copy(h_hbm.at[iv], hbuf, gsems[b])
            pltpu.async_copy(c_hbm.at[iv], cbuf, gsems[b])

        for b in range(_NBUF):
            issue_gather(b, b)

        def outer(g, carry):
            for b in range(_NBUF):
                j = g * _NBUF + b
                hbuf, cbuf = bufs(b)
                iv = idx_v.at[pl.ds(j * _CHUNK, _CHUNK)]
                pltpu.make_async_copy(h_hbm.at[iv], hbuf, gsems[b]).wait()
                pltpu.make_async_copy(c_hbm.at[iv], cbuf, gsems[b]).wait()
                base = wbase + j * _CHUNK
                cph = pltpu.async_copy(
                    hbuf, hout_hbm.at[pl.ds(base, _CHUNK)], wsems[b])
                cpc = pltpu.async_copy(
                    cbuf, cout_hbm.at[pl.ds(base, _CHUNK)], wsems[b])
                cph.wait()
                cpc.wait()

                @pl.when(j + _NBUF < _CPW)
                def _():
                    issue_gather(j + _NBUF, b)
            return carry

        lax.fori_loop(0, _CPW // _NBUF, outer, 0)

    @pl.when(wid < _GPW)
    def _():
        run(idx0_hbm, h0_out, c0_out, wid)

    @pl.when(wid >= _GPW)
    def _():
        run(idx1_hbm, h1_out, c1_out, wid - _GPW)


@functools.cache
def _make_sc_gather():
    mail = jax.ShapeDtypeStruct((_NSP, H), jnp.float32)
    return functools.partial(
        pl.kernel,
        mesh=plsc.VectorSubcoreMesh(core_axis_name="c", subcore_axis_name="s"),
        out_type=(mail, mail, mail, mail),
        scratch_types=[
            pltpu.VMEM((_CPW * _CHUNK,), jnp.int32),
            pltpu.VMEM((_NBUF * _CHUNK, H), jnp.float32),
            pltpu.VMEM((_NBUF * _CHUNK, H), jnp.float32),
            pltpu.SemaphoreType.DMA,
            pltpu.SemaphoreType.DMA,
            pltpu.SemaphoreType.DMA,
            pltpu.SemaphoreType.DMA,
        ],
    )(_gather_body)


_BN = 800
_BPS = _NSLAB // _BN


def _dense_body(x_ref, h1_ref, h2_ref, c1_ref, c2_ref, tf_ref,
                wx_ref, wh_ref, wf_ref, biou_ref, bious_ref, bf_ref,
                hacc_ref, cacc_ref, h_out_ref, c_out_ref):
    del hacc_ref, cacc_ref
    x = x_ref[...]
    h1 = h1_ref[...]
    h2 = h2_ref[...]
    wh = wh_ref[...]
    wf = wf_ref[...]
    iou_both = (
        jnp.dot(x, wx_ref[...], preferred_element_type=jnp.float32)
        + jnp.dot(h1, wh[:H], preferred_element_type=jnp.float32)
        + jnp.dot(h2, wh[H:], preferred_element_type=jnp.float32)
    )
    f = jax.nn.sigmoid(
        jnp.dot(h1, wf[:H], preferred_element_type=jnp.float32)
        + jnp.dot(h2, wf[H:], preferred_element_type=jnp.float32)
        + bf_ref[...]
    )
    c1 = c1_ref[...]
    c2 = c2_ref[...]
    c_n = f[:, :H] * c1 + f[:, H:2 * H] * c2
    c_sm = f[:, 2 * H:3 * H] * c1 + f[:, 3 * H:] * c2
    iou_n = iou_both[:, :3 * H] + biou_ref[...]
    iou_sm = iou_both[:, 3 * H:] + bious_ref[...]
    tm = tf_ref[...]
    iou = iou_n + tm * (iou_sm - iou_n)
    c_r = c_n + tm * (c_sm - c_n)
    c_out = jax.nn.sigmoid(iou[:, :H]) * jnp.tanh(iou[:, 2 * H:]) + c_r
    c_out_ref[...] = c_out
    h_out_ref[...] = jax.nn.sigmoid(iou[:, H:2 * H]) * jnp.tanh(c_out)


def _dense_call(s, x, mail, tf, wx, wh, wf, biou, bious, bf, hacc, cacc):
    h0, h1, c0, c1 = mail
    off = s * _BPS
    glob_spec = lambda w: pl.BlockSpec((_BN, w), lambda i: (i + off, 0))
    loc_spec = pl.BlockSpec((_BN, H), lambda i: (i, 0))
    full_spec = lambda a, b: pl.BlockSpec((a, b), lambda i: (0, 0))
    any_spec = pl.BlockSpec(memory_space=pl.ANY)
    return pl.pallas_call(
        _dense_body,
        grid=(_BPS,),
        in_specs=[
            glob_spec(X),
            loc_spec,
            loc_spec,
            loc_spec,
            loc_spec,
            glob_spec(1),
            full_spec(X, 6 * H),
            full_spec(2 * H, 6 * H),
            full_spec(2 * H, 4 * H),
            full_spec(1, 3 * H),
            full_spec(1, 3 * H),
            full_spec(1, 4 * H),
            any_spec,
            any_spec,
        ],
        out_specs=[pl.BlockSpec((_BN, H), lambda i: (i + off, 0)),
                   pl.BlockSpec((_BN, H), lambda i: (i + off, 0))],
        out_shape=[
            jax.ShapeDtypeStruct((N, H), jnp.float32),
            jax.ShapeDtypeStruct((N, H), jnp.float32),
        ],
        input_output_aliases={12: 0, 13: 1},
    )(x, h0, h1, c0, c1, tf, wx, wh, wf, biou, bious, bf, hacc, cacc)


def kernel(x, h_src, c_src, child_idx, t, W_iou, U_iou, b_iou, U_f_w, U_f_b,
           W_iou_s, U_iou_s, b_iou_s, U_f_s_w, U_f_s_b):
    def slab_idx(col):
        pad = jnp.zeros((_K, _NSP - _NSLAB), dtype=jnp.int32)
        return jnp.concatenate(
            [child_idx[:, col].reshape(_K, _NSLAB), pad], axis=1)

    idx0 = slab_idx(0)
    idx1 = slab_idx(1)

    sc_gather = _make_sc_gather()
    mails = [sc_gather(h_src, c_src, idx0[s], idx1[s]) for s in range(_K)]

    wx = jnp.concatenate([W_iou, W_iou_s], axis=1)
    wh = jnp.concatenate(
        [U_iou, jnp.concatenate([U_iou_s, U_iou_s], axis=0)], axis=1)
    z = jnp.zeros((H, H), dtype=jnp.float32)
    ufs_bd = jnp.block([[U_f_s_w, z], [z, U_f_s_w]])
    wf = jnp.concatenate([U_f_w, ufs_bd], axis=1)
    bf = jnp.concatenate([U_f_b, U_f_s_b, U_f_s_b]).reshape(1, 4 * H)
    tf = (t == 1).astype(jnp.float32).reshape(N, 1)

    hacc = None
    cacc = None
    for s in range(_K):
        if s == 0:
            hacc = jnp.zeros((N, H), dtype=jnp.float32)
            cacc = jnp.zeros((N, H), dtype=jnp.float32)
        hacc, cacc = _dense_call(s, x, mails[s], tf, wx, wh, wf,
                                 b_iou, b_iou_s, bf, hacc, cacc)
    return (hacc, cacc)
